# baseline (speedup 1.0000x reference)
import functools

import jax
import jax.numpy as jnp
from jax import lax
from jax.experimental import pallas as pl
from jax.experimental.pallas import tpu as pltpu, tpu_sc as plsc

N_NODES = 10000
D = 128
E = 320000

NC = 2
NS = 16
NW = NC * NS
EK = 128
E_PAD = ((E + NW * EK - 1) // (NW * EK)) * (NW * EK)
E_PER_W = E_PAD // NW
N_BLK = E_PER_W // EK
N_ACC = 10112
ROWS_PER_SUB = N_ACC // NS
GRP = 4
EP_ROWS = 16


_DNUMS = lax.GatherDimensionNumbers(
    offset_dims=(), collapsed_slice_dims=(0,), start_index_map=(0,))


def _scale_rows(rows, ebuf, arow):
    def scale16(eb, _):
        av = lax.bitcast_convert_type(ebuf[arow, pl.ds(eb * 16, 16)], jnp.float32)
        for l in range(16):
            a = lax.gather(av, jnp.full((16, 1), l, jnp.int32), _DNUMS, (1,),
                           mode=lax.GatherScatterMode.PROMISE_IN_BOUNDS)
            e = eb * 16 + l
            for j in range(D // 16):
                rows[e, pl.ds(j * 16, 16)] = rows[e, pl.ds(j * 16, 16)] * a
        return 0

    lax.fori_loop(0, EK // 16, scale16, 0)


def _sc_scatter_spmm(x, epack):
    mesh = plsc.VectorSubcoreMesh(core_axis_name="c", subcore_axis_name="s")

    @functools.partial(
        pl.kernel,
        mesh=mesh,
        out_type=jax.ShapeDtypeStruct((NC, N_ACC, D), jnp.float32),
        scratch_types=[
            pltpu.VMEM((3, EK), jnp.int32),
            pltpu.VMEM((EK, D), jnp.float32),
            pltpu.VMEM_SHARED((N_ACC, D), jnp.float32),
            pltpu.SemaphoreType.DMA,
        ],
    )
    def k(x_hbm, ep_hbm, out_hbm, ebuf0, rows0, accum, gs0):
        c = lax.axis_index("c")
        s = lax.axis_index("s")
        wid = s * NC + c
        cid0 = wid * N_BLK

        zero = jnp.zeros((16,), jnp.float32)

        def zrow(i, _):
            for j in range(D // 16):
                rows0[i, pl.ds(j * 16, 16)] = zero
            return 0

        lax.fori_loop(0, EK, zrow, 0)
        nfull = ROWS_PER_SUB // EK
        rem = ROWS_PER_SUB - nfull * EK
        for i in range(nfull):
            pltpu.sync_copy(rows0, accum.at[pl.ds(s * ROWS_PER_SUB + i * EK, EK)])
        if rem:
            pltpu.sync_copy(rows0.at[pl.ds(0, rem)],
                            accum.at[pl.ds(s * ROWS_PER_SUB + nfull * EK, rem)])
        plsc.subcore_barrier()

        def chunk(g, _):
            pltpu.sync_copy(ep_hbm.at[cid0 + g], ebuf0)
            pltpu.async_copy(x_hbm.at[ebuf0.at[0]], rows0, gs0).wait()
            _scale_rows(rows0, ebuf0, 2)
            pltpu.sync_copy(rows0, accum.at[ebuf0.at[1]], add=True)
            return 0

        lax.fori_loop(0, N_BLK, chunk, 0)
        plsc.subcore_barrier()

        pltpu.sync_copy(accum.at[pl.ds(s * ROWS_PER_SUB, ROWS_PER_SUB)],
                        out_hbm.at[c, pl.ds(s * ROWS_PER_SUB, ROWS_PER_SUB)])

    return k(x, epack)


def _dense_body(x_ref, s0_ref, s1_ref, w_ref, b_ref, o_ref):
    feat = 0.55 * x_ref[...] + 0.95 * (s0_ref[...] + s1_ref[...])
    o_ref[...] = jnp.dot(feat, w_ref[...],
                         preferred_element_type=jnp.float32) + b_ref[...]


def _dense(x, s0, s1, W, b):
    BM = 1000
    grid = (N_NODES // BM,)
    return pl.pallas_call(
        _dense_body,
        grid=grid,
        in_specs=[
            pl.BlockSpec((BM, D), lambda i: (i, 0)),
            pl.BlockSpec((BM, D), lambda i: (i, 0)),
            pl.BlockSpec((BM, D), lambda i: (i, 0)),
            pl.BlockSpec((D, D), lambda i: (0, 0)),
            pl.BlockSpec((1, D), lambda i: (0, 0)),
        ],
        out_specs=pl.BlockSpec((BM, D), lambda i: (i, 0)),
        out_shape=jax.ShapeDtypeStruct((N_NODES, D), jnp.float32),
    )(x, s0, s1, W, b)


def kernel(x, edge_indices, edge_attr, W, b):
    pad = E_PAD - E
    src = jnp.concatenate(
        [edge_indices[0].astype(jnp.int32), jnp.zeros((pad,), jnp.int32)])
    dst = jnp.concatenate(
        [edge_indices[1].astype(jnp.int32), jnp.zeros((pad,), jnp.int32)])
    attr_bits = jnp.concatenate(
        [edge_attr.astype(jnp.float32), jnp.zeros((pad,), jnp.float32)]
    ).view(jnp.int32)

    nchunks = E_PAD // EK
    epack = jnp.stack([src.reshape(nchunks, EK),
                       dst.reshape(nchunks, EK),
                       attr_bits.reshape(nchunks, EK)], axis=1)

    parts = _sc_scatter_spmm(x, epack)
    return _dense(x, parts[0], parts[1], W, b.reshape(1, D))


# speedup vs baseline: 1.0096x; 1.0096x over previous
import functools

import jax
import jax.numpy as jnp
from jax import lax
from jax.experimental import pallas as pl
from jax.experimental.pallas import tpu as pltpu, tpu_sc as plsc

N_NODES = 10000
D = 128
E = 320000

NC = 2
NS = 16
NW = NC * NS
EK = 128
E_PAD = ((E + NW * EK - 1) // (NW * EK)) * (NW * EK)
E_PER_W = E_PAD // NW
N_BLK = E_PER_W // EK
N_ACC = 10112
ROWS_PER_SUB = N_ACC // NS
GRP = 4
EP_ROWS = 16


_DNUMS = lax.GatherDimensionNumbers(
    offset_dims=(), collapsed_slice_dims=(0,), start_index_map=(0,))


def _scale_rows(rows, ebuf, arow):
    def scale16(eb, _):
        av = lax.bitcast_convert_type(ebuf[arow, pl.ds(eb * 16, 16)], jnp.float32)
        for l in range(16):
            a = lax.gather(av, jnp.full((16, 1), l, jnp.int32), _DNUMS, (1,),
                           mode=lax.GatherScatterMode.PROMISE_IN_BOUNDS)
            e = eb * 16 + l
            for j in range(D // 16):
                rows[e, pl.ds(j * 16, 16)] = rows[e, pl.ds(j * 16, 16)] * a
        return 0

    lax.fori_loop(0, EK // 16, scale16, 0)


def _sc_scatter_spmm(x, epack):
    mesh = plsc.VectorSubcoreMesh(core_axis_name="c", subcore_axis_name="s")

    @functools.partial(
        pl.kernel,
        mesh=mesh,
        out_type=jax.ShapeDtypeStruct((NC, N_ACC, D), jnp.float32),
        scratch_types=[
            pltpu.VMEM((3, EK), jnp.int32),
            pltpu.VMEM((EK, D), jnp.float32),
            pltpu.VMEM_SHARED((N_ACC, D), jnp.float32),
            pltpu.SemaphoreType.DMA,
            pltpu.SemaphoreType.DMA,
            pltpu.SemaphoreType.DMA,
            pltpu.SemaphoreType.DMA,
        ],
    )
    def k(x_hbm, ep_hbm, out_hbm, ebuf0, rows0, accum, gs0, gs1, gs2, gs3):
        c = lax.axis_index("c")
        s = lax.axis_index("s")
        wid = s * NC + c
        cid0 = wid * N_BLK

        zero = jnp.zeros((16,), jnp.float32)

        def zrow(i, _):
            for j in range(D // 16):
                rows0[i, pl.ds(j * 16, 16)] = zero
            return 0

        lax.fori_loop(0, EK, zrow, 0)
        nfull = ROWS_PER_SUB // EK
        rem = ROWS_PER_SUB - nfull * EK
        for i in range(nfull):
            pltpu.sync_copy(rows0, accum.at[pl.ds(s * ROWS_PER_SUB + i * EK, EK)])
        if rem:
            pltpu.sync_copy(rows0.at[pl.ds(0, rem)],
                            accum.at[pl.ds(s * ROWS_PER_SUB + nfull * EK, rem)])
        plsc.subcore_barrier()

        def chunk(g, _):
            pltpu.sync_copy(ep_hbm.at[cid0 + g], ebuf0)
            gsems = (gs0, gs1, gs2, gs3)
            q = EK // 4
            cps = [pltpu.async_copy(
                       x_hbm.at[ebuf0.at[0, pl.ds(i * q, q)]],
                       rows0.at[pl.ds(i * q, q)], gsems[i])
                   for i in range(4)]
            for cp in cps:
                cp.wait()
            _scale_rows(rows0, ebuf0, 2)
            pltpu.sync_copy(rows0, accum.at[ebuf0.at[1]], add=True)
            return 0

        lax.fori_loop(0, N_BLK, chunk, 0)
        plsc.subcore_barrier()

        pltpu.sync_copy(accum.at[pl.ds(s * ROWS_PER_SUB, ROWS_PER_SUB)],
                        out_hbm.at[c, pl.ds(s * ROWS_PER_SUB, ROWS_PER_SUB)])

    return k(x, epack)


def _dense_body(x_ref, s0_ref, s1_ref, w_ref, b_ref, o_ref):
    feat = 0.55 * x_ref[...] + 0.95 * (s0_ref[...] + s1_ref[...])
    o_ref[...] = jnp.dot(feat, w_ref[...],
                         preferred_element_type=jnp.float32) + b_ref[...]


def _dense(x, s0, s1, W, b):
    BM = 1000
    grid = (N_NODES // BM,)
    return pl.pallas_call(
        _dense_body,
        grid=grid,
        in_specs=[
            pl.BlockSpec((BM, D), lambda i: (i, 0)),
            pl.BlockSpec((BM, D), lambda i: (i, 0)),
            pl.BlockSpec((BM, D), lambda i: (i, 0)),
            pl.BlockSpec((D, D), lambda i: (0, 0)),
            pl.BlockSpec((1, D), lambda i: (0, 0)),
        ],
        out_specs=pl.BlockSpec((BM, D), lambda i: (i, 0)),
        out_shape=jax.ShapeDtypeStruct((N_NODES, D), jnp.float32),
    )(x, s0, s1, W, b)


def kernel(x, edge_indices, edge_attr, W, b):
    pad = E_PAD - E
    src = jnp.concatenate(
        [edge_indices[0].astype(jnp.int32), jnp.zeros((pad,), jnp.int32)])
    dst = jnp.concatenate(
        [edge_indices[1].astype(jnp.int32), jnp.zeros((pad,), jnp.int32)])
    attr_bits = jnp.concatenate(
        [edge_attr.astype(jnp.float32), jnp.zeros((pad,), jnp.float32)]
    ).view(jnp.int32)

    nchunks = E_PAD // EK
    epack = jnp.stack([src.reshape(nchunks, EK),
                       dst.reshape(nchunks, EK),
                       attr_bits.reshape(nchunks, EK)], axis=1)

    parts = _sc_scatter_spmm(x, epack)
    return _dense(x, parts[0], parts[1], W, b.reshape(1, D))
